# baseline (speedup 1.0000x reference)
import functools

import jax
import jax.numpy as jnp
from jax import lax
from jax.experimental import pallas as pl
from jax.experimental.pallas import tpu as pltpu
from jax.experimental.pallas import tpu_sc as plsc

_C, _H, _W = 96, 512, 512
_NW = 32
_RW = _H // _NW
_NCH = _W // 16


def _compute(mask_v, src, dst):
    def row_body(r, _):
        for j in range(_NCH):
            sl = pl.ds(j * 16, 16)
            m = mask_v[r, sl] > 0.9
            dst[r, sl] = jnp.where(m, src[r, sl], 0.0)
        return 0

    lax.fori_loop(0, _RW, row_body, 0, unroll=False)


def _sc_body(c0, k, img_hbm, out_hbm, mask_v, i0, i1, o0, o1,
             smask, si0, si1, so0, so1):
    nc = 2
    wid = lax.axis_index("s") * nc + lax.axis_index("c")
    r0 = wid * _RW
    rows = pl.ds(r0, _RW)

    pltpu.async_copy(img_hbm.at[_C - 1, rows, :], mask_v, smask).wait()
    pltpu.async_copy(img_hbm.at[c0, rows, :], i0, si0)

    def pair(p, _):
        c = c0 + 2 * p
        pltpu.async_copy(img_hbm.at[c + 1, rows, :], i1, si1)
        pltpu.make_async_copy(img_hbm.at[c, rows, :], i0, si0).wait()

        @pl.when(p > 0)
        def _():
            pltpu.make_async_copy(o0, out_hbm.at[c - c0 - 2, rows, :], so0).wait()

        _compute(mask_v, i0, o0)
        pltpu.async_copy(o0, out_hbm.at[c - c0, rows, :], so0)

        @pl.when(p + 1 < k // 2)
        def _():
            pltpu.async_copy(img_hbm.at[c + 2, rows, :], i0, si0)

        pltpu.make_async_copy(img_hbm.at[c + 1, rows, :], i1, si1).wait()

        @pl.when(p > 0)
        def _():
            pltpu.make_async_copy(o1, out_hbm.at[c - c0 - 1, rows, :], so1).wait()

        _compute(mask_v, i1, o1)
        pltpu.async_copy(o1, out_hbm.at[c - c0 + 1, rows, :], so1)
        return 0

    lax.fori_loop(0, k // 2, pair, 0, unroll=False)
    last = c0 + k - 2
    pltpu.make_async_copy(o0, out_hbm.at[last - c0, rows, :], so0).wait()
    pltpu.make_async_copy(o1, out_hbm.at[last - c0 + 1, rows, :], so1).wait()


def _make_sc_select(c0, k):
    mesh = plsc.VectorSubcoreMesh(core_axis_name="c", subcore_axis_name="s")
    return pl.kernel(
        functools.partial(_sc_body, c0, k),
        out_type=jax.ShapeDtypeStruct((k, _H, _W), jnp.float32),
        mesh=mesh,
        scratch_types=[
            pltpu.VMEM((_RW, _W), jnp.float32),
            pltpu.VMEM((_RW, _W), jnp.float32),
            pltpu.VMEM((_RW, _W), jnp.float32),
            pltpu.VMEM((_RW, _W), jnp.float32),
            pltpu.VMEM((_RW, _W), jnp.float32),
            pltpu.SemaphoreType.DMA,
            pltpu.SemaphoreType.DMA,
            pltpu.SemaphoreType.DMA,
            pltpu.SemaphoreType.DMA,
            pltpu.SemaphoreType.DMA,
        ],
    )


def kernel(img):
    return _make_sc_select(0, _C)(img)


# speedup vs baseline: 1.0506x; 1.0506x over previous
import functools

import jax
import jax.numpy as jnp
from jax import lax
from jax.experimental import pallas as pl
from jax.experimental.pallas import tpu as pltpu
from jax.experimental.pallas import tpu_sc as plsc

_C, _H, _W = 96, 512, 512
_NW = 32
_RW = _H // _NW
_NCH = _W // 16


def _compute(mask_v, src, dst):
    def row_body(r, _):
        for j in range(_NCH):
            sl = pl.ds(j * 16, 16)
            m = mask_v[r, sl] > 0.9
            dst[r, sl] = jnp.where(m, src[r, sl], 0.0)
        return 0

    lax.fori_loop(0, _RW, row_body, 0, unroll=False)


def _sc_body(c0, k, img_hbm, out_hbm, mask_v, i0, i1, o0, o1,
             smask, si0, si1, so0, so1):
    nc = 2
    wid = lax.axis_index("s") * nc + lax.axis_index("c")
    r0 = wid * _RW
    rows = pl.ds(r0, _RW)

    pltpu.async_copy(img_hbm.at[_C - 1, rows, :], mask_v, smask).wait()
    pltpu.async_copy(img_hbm.at[c0, rows, :], i0, si0)

    def pair(p, _):
        c = c0 + 2 * p
        pltpu.async_copy(img_hbm.at[c + 1, rows, :], i1, si1)
        pltpu.make_async_copy(img_hbm.at[c, rows, :], i0, si0).wait()

        @pl.when(p > 0)
        def _():
            pltpu.make_async_copy(o0, out_hbm.at[c - c0 - 2, rows, :], so0).wait()

        _compute(mask_v, i0, o0)
        pltpu.async_copy(o0, out_hbm.at[c - c0, rows, :], so0)

        @pl.when(p + 1 < k // 2)
        def _():
            pltpu.async_copy(img_hbm.at[c + 2, rows, :], i0, si0)

        pltpu.make_async_copy(img_hbm.at[c + 1, rows, :], i1, si1).wait()

        @pl.when(p > 0)
        def _():
            pltpu.make_async_copy(o1, out_hbm.at[c - c0 - 1, rows, :], so1).wait()

        _compute(mask_v, i1, o1)
        pltpu.async_copy(o1, out_hbm.at[c - c0 + 1, rows, :], so1)
        return 0

    lax.fori_loop(0, k // 2, pair, 0, unroll=False)
    last = c0 + k - 2
    pltpu.make_async_copy(o0, out_hbm.at[last - c0, rows, :], so0).wait()
    pltpu.make_async_copy(o1, out_hbm.at[last - c0 + 1, rows, :], so1).wait()


def _make_sc_select(c0, k):
    mesh = plsc.VectorSubcoreMesh(core_axis_name="c", subcore_axis_name="s")
    return pl.kernel(
        functools.partial(_sc_body, c0, k),
        out_type=jax.ShapeDtypeStruct((k, _H, _W), jnp.float32),
        mesh=mesh,
        scratch_types=[
            pltpu.VMEM((_RW, _W), jnp.float32),
            pltpu.VMEM((_RW, _W), jnp.float32),
            pltpu.VMEM((_RW, _W), jnp.float32),
            pltpu.VMEM((_RW, _W), jnp.float32),
            pltpu.VMEM((_RW, _W), jnp.float32),
            pltpu.SemaphoreType.DMA,
            pltpu.SemaphoreType.DMA,
            pltpu.SemaphoreType.DMA,
            pltpu.SemaphoreType.DMA,
            pltpu.SemaphoreType.DMA,
        ],
    )


_K = 28
_BC = 4


def _tc_select_block(x_ref, m_ref, o_ref):
    mask = m_ref[...] > 0.9
    o_ref[...] = jnp.where(mask, x_ref[...], 0.0)


def _tc_select(img):
    nblk = (_C - _K) // _BC
    return pl.pallas_call(
        _tc_select_block,
        grid=(nblk,),
        in_specs=[
            pl.BlockSpec((_BC, _H, _W), lambda i: (i + _K // _BC, 0, 0)),
            pl.BlockSpec((1, _H, _W), lambda i: (_C - 1, 0, 0)),
        ],
        out_specs=pl.BlockSpec((_BC, _H, _W), lambda i: (i, 0, 0)),
        out_shape=jax.ShapeDtypeStruct((_C - _K, _H, _W), jnp.float32),
    )(img, img)


def kernel(img):
    lo = _make_sc_select(0, _K)(img)
    hi = _tc_select(img)
    return jnp.concatenate([lo, hi], axis=0)


# speedup vs baseline: 2.4388x; 2.3214x over previous
import functools

import jax
import jax.numpy as jnp
from jax import lax
from jax.experimental import pallas as pl
from jax.experimental.pallas import tpu as pltpu
from jax.experimental.pallas import tpu_sc as plsc

_C, _H, _W = 96, 512, 512
_NW = 32
_RW = _H // _NW
_NCH = _W // 16


def _compute(mask_v, src, dst):
    def row_body(r, _):
        for j in range(_NCH):
            sl = pl.ds(j * 16, 16)
            m = mask_v[r, sl] > 0.9
            dst[r, sl] = jnp.where(m, src[r, sl], 0.0)
        return 0

    lax.fori_loop(0, _RW, row_body, 0, unroll=False)


def _sc_body(c0, k, img_hbm, out_hbm, mask_v, i0, i1, o0, o1,
             smask, si0, si1, so0, so1):
    nc = 2
    wid = lax.axis_index("s") * nc + lax.axis_index("c")
    r0 = wid * _RW
    rows = pl.ds(r0, _RW)

    pltpu.async_copy(img_hbm.at[_C - 1, rows, :], mask_v, smask).wait()
    pltpu.async_copy(img_hbm.at[c0, rows, :], i0, si0)

    def pair(p, _):
        c = c0 + 2 * p
        pltpu.async_copy(img_hbm.at[c + 1, rows, :], i1, si1)
        pltpu.make_async_copy(img_hbm.at[c, rows, :], i0, si0).wait()

        @pl.when(p > 0)
        def _():
            pltpu.make_async_copy(o0, out_hbm.at[c - c0 - 2, rows, :], so0).wait()

        _compute(mask_v, i0, o0)
        pltpu.async_copy(o0, out_hbm.at[c - c0, rows, :], so0)

        @pl.when(p + 1 < k // 2)
        def _():
            pltpu.async_copy(img_hbm.at[c + 2, rows, :], i0, si0)

        pltpu.make_async_copy(img_hbm.at[c + 1, rows, :], i1, si1).wait()

        @pl.when(p > 0)
        def _():
            pltpu.make_async_copy(o1, out_hbm.at[c - c0 - 1, rows, :], so1).wait()

        _compute(mask_v, i1, o1)
        pltpu.async_copy(o1, out_hbm.at[c - c0 + 1, rows, :], so1)
        return 0

    lax.fori_loop(0, k // 2, pair, 0, unroll=False)
    last = c0 + k - 2
    pltpu.make_async_copy(o0, out_hbm.at[last - c0, rows, :], so0).wait()
    pltpu.make_async_copy(o1, out_hbm.at[last - c0 + 1, rows, :], so1).wait()


def _make_sc_select(c0, k):
    mesh = plsc.VectorSubcoreMesh(core_axis_name="c", subcore_axis_name="s")
    return pl.kernel(
        functools.partial(_sc_body, c0, k),
        out_type=jax.ShapeDtypeStruct((k, _H, _W), jnp.float32),
        mesh=mesh,
        scratch_types=[
            pltpu.VMEM((_RW, _W), jnp.float32),
            pltpu.VMEM((_RW, _W), jnp.float32),
            pltpu.VMEM((_RW, _W), jnp.float32),
            pltpu.VMEM((_RW, _W), jnp.float32),
            pltpu.VMEM((_RW, _W), jnp.float32),
            pltpu.SemaphoreType.DMA,
            pltpu.SemaphoreType.DMA,
            pltpu.SemaphoreType.DMA,
            pltpu.SemaphoreType.DMA,
            pltpu.SemaphoreType.DMA,
        ],
    )


_BC = 8


def _tc_select_block(x_ref, m_ref, o_ref):
    mask = m_ref[...] > 0.9
    o_ref[...] = jnp.where(mask, x_ref[...], 0.0)


def kernel(img):
    return pl.pallas_call(
        _tc_select_block,
        grid=(_C // _BC,),
        in_specs=[
            pl.BlockSpec((_BC, _H, _W), lambda i: (i, 0, 0)),
            pl.BlockSpec((1, _H, _W), lambda i: (_C - 1, 0, 0)),
        ],
        out_specs=pl.BlockSpec((_BC, _H, _W), lambda i: (i, 0, 0)),
        out_shape=jax.ShapeDtypeStruct((_C, _H, _W), jnp.float32),
    )(img, img)


# speedup vs baseline: 2.4712x; 1.0133x over previous
import functools

import jax
import jax.numpy as jnp
from jax import lax
from jax.experimental import pallas as pl
from jax.experimental.pallas import tpu as pltpu
from jax.experimental.pallas import tpu_sc as plsc

_C, _H, _W = 96, 512, 512
_NW = 32
_RW = _H // _NW
_NCH = _W // 16


def _compute(mask_v, src, dst):
    def row_body(r, _):
        for j in range(_NCH):
            sl = pl.ds(j * 16, 16)
            m = mask_v[r, sl] > 0.9
            dst[r, sl] = jnp.where(m, src[r, sl], 0.0)
        return 0

    lax.fori_loop(0, _RW, row_body, 0, unroll=False)


def _sc_body(c0, k, img_hbm, out_hbm, mask_v, i0, i1, o0, o1,
             smask, si0, si1, so0, so1):
    nc = 2
    wid = lax.axis_index("s") * nc + lax.axis_index("c")
    r0 = wid * _RW
    rows = pl.ds(r0, _RW)

    pltpu.async_copy(img_hbm.at[_C - 1, rows, :], mask_v, smask).wait()
    pltpu.async_copy(img_hbm.at[c0, rows, :], i0, si0)

    def pair(p, _):
        c = c0 + 2 * p
        pltpu.async_copy(img_hbm.at[c + 1, rows, :], i1, si1)
        pltpu.make_async_copy(img_hbm.at[c, rows, :], i0, si0).wait()

        @pl.when(p > 0)
        def _():
            pltpu.make_async_copy(o0, out_hbm.at[c - c0 - 2, rows, :], so0).wait()

        _compute(mask_v, i0, o0)
        pltpu.async_copy(o0, out_hbm.at[c - c0, rows, :], so0)

        @pl.when(p + 1 < k // 2)
        def _():
            pltpu.async_copy(img_hbm.at[c + 2, rows, :], i0, si0)

        pltpu.make_async_copy(img_hbm.at[c + 1, rows, :], i1, si1).wait()

        @pl.when(p > 0)
        def _():
            pltpu.make_async_copy(o1, out_hbm.at[c - c0 - 1, rows, :], so1).wait()

        _compute(mask_v, i1, o1)
        pltpu.async_copy(o1, out_hbm.at[c - c0 + 1, rows, :], so1)
        return 0

    lax.fori_loop(0, k // 2, pair, 0, unroll=False)
    last = c0 + k - 2
    pltpu.make_async_copy(o0, out_hbm.at[last - c0, rows, :], so0).wait()
    pltpu.make_async_copy(o1, out_hbm.at[last - c0 + 1, rows, :], so1).wait()


def _make_sc_select(c0, k):
    mesh = plsc.VectorSubcoreMesh(core_axis_name="c", subcore_axis_name="s")
    return pl.kernel(
        functools.partial(_sc_body, c0, k),
        out_type=jax.ShapeDtypeStruct((k, _H, _W), jnp.float32),
        mesh=mesh,
        scratch_types=[
            pltpu.VMEM((_RW, _W), jnp.float32),
            pltpu.VMEM((_RW, _W), jnp.float32),
            pltpu.VMEM((_RW, _W), jnp.float32),
            pltpu.VMEM((_RW, _W), jnp.float32),
            pltpu.VMEM((_RW, _W), jnp.float32),
            pltpu.SemaphoreType.DMA,
            pltpu.SemaphoreType.DMA,
            pltpu.SemaphoreType.DMA,
            pltpu.SemaphoreType.DMA,
            pltpu.SemaphoreType.DMA,
        ],
    )


_BC = 12


def _tc_select_block(x_ref, m_ref, o_ref):
    mask = m_ref[...] > 0.9
    o_ref[...] = jnp.where(mask, x_ref[...], 0.0)


def kernel(img):
    return pl.pallas_call(
        _tc_select_block,
        grid=(_C // _BC,),
        in_specs=[
            pl.BlockSpec((_BC, _H, _W), lambda i: (i, 0, 0)),
            pl.BlockSpec((1, _H, _W), lambda i: (_C - 1, 0, 0)),
        ],
        out_specs=pl.BlockSpec((_BC, _H, _W), lambda i: (i, 0, 0)),
        out_shape=jax.ShapeDtypeStruct((_C, _H, _W), jnp.float32),
    )(img, img)
